# baseline (speedup 1.0000x reference)
import jax
import jax.numpy as jnp
from jax.experimental import pallas as pl
from jax.experimental.pallas import tpu as pltpu

_N = 10000
_D = 128
_CHUNK = 200
_NBUF = 5
_NCHUNK = _N // _CHUNK


def _gcn_kernel(a_hbm, x_ref, w_ref, b_ref, out_ref, buf, s_ref, sems):
    def _copy(c, slot):
        return pltpu.make_async_copy(
            a_hbm.at[pl.ds(c * _CHUNK, _CHUNK), :],
            buf.at[slot],
            sems.at[slot],
        )

    for k in range(_NBUF):
        _copy(k, k).start()

    s_ref[...] = jnp.dot(x_ref[...], w_ref[...],
                         preferred_element_type=jnp.float32,
                         precision=jax.lax.Precision.DEFAULT)

    def _step(c, carry):
        slot = jax.lax.rem(c, _NBUF)
        _copy(c, slot).wait()
        out_ref[pl.ds(c * _CHUNK, _CHUNK), :] = (
            jnp.dot(buf[slot], s_ref[...],
                    preferred_element_type=jnp.float32,
                    precision=jax.lax.Precision.DEFAULT)
            + b_ref[...]
        )

        @pl.when(c + _NBUF < _NCHUNK)
        def _():
            _copy(c + _NBUF, slot).start()

        return carry

    jax.lax.fori_loop(0, _NCHUNK, _step, 0)


def kernel(x, edge_index, weight, bias):
    return pl.pallas_call(
        _gcn_kernel,
        in_specs=[
            pl.BlockSpec(memory_space=pltpu.MemorySpace.HBM),
            pl.BlockSpec(memory_space=pltpu.MemorySpace.VMEM),
            pl.BlockSpec(memory_space=pltpu.MemorySpace.VMEM),
            pl.BlockSpec(memory_space=pltpu.MemorySpace.VMEM),
        ],
        out_specs=pl.BlockSpec(memory_space=pltpu.MemorySpace.VMEM),
        out_shape=jax.ShapeDtypeStruct((_N, _D), jnp.float32),
        scratch_shapes=[
            pltpu.VMEM((_NBUF, _CHUNK, _N), jnp.float32),
            pltpu.VMEM((_N, _D), jnp.float32),
            pltpu.SemaphoreType.DMA((_NBUF,)),
        ],
    )(edge_index, x, weight, bias.reshape(1, _D))
